# Initial kernel scaffold; baseline (speedup 1.0000x reference)
import jax
import jax.numpy as jnp
from jax.experimental import pallas as pl


def kernel(scalar_rep, vector_rep, edge_index, frames, W_vd, W_vdf, W_so, b_so, W_vu, W_vos, b_vos):
    raise NotImplementedError("write your pallas kernel here")


# speedup vs baseline: 1.3391x; 1.3391x over previous
import functools

import jax
import jax.numpy as jnp
from jax import lax
from jax.experimental import pallas as pl
from jax.experimental.pallas import tpu as pltpu
from jax.experimental.pallas import tpu_sc as plsc

_NC = 2
_NS = 16
_NW = _NC * _NS
_L = 16

_EPS = 1e-8
_PREC = lax.Precision.HIGHEST


def _vdf_body(vr_ref, w_ref, out_ref):
    out_ref[...] = lax.dot_general(
        vr_ref[...], w_ref[...], (((1,), (0,)), ((), ())),
        preferred_element_type=jnp.float32, precision=_PREC)


def _vdf_stage(vr2, w1p, blk):
    n = vr2.shape[0]
    return pl.pallas_call(
        _vdf_body,
        grid=(n // blk,),
        in_specs=[
            pl.BlockSpec((blk, 48), lambda i: (i, 0)),
            pl.BlockSpec((48, 16), lambda i: (0, 0)),
        ],
        out_specs=pl.BlockSpec((blk, 16), lambda i: (i, 0)),
        out_shape=jax.ShapeDtypeStruct((n, 16), jnp.float32),
    )(vr2, w1p)


def _sc_body(n_nodes, n_groups_total, row_hbm, col_hbm, fr_hbm, vdf_hbm,
             out_hbm, rbuf, cbuf, fbuf, vgbuf, outbuf, zbuf, acc_sh, sem):
    cid = lax.axis_index("c")
    sid = lax.axis_index("s")
    wid = sid * _NC + cid

    octets = n_nodes // 8
    base_o = octets // _NS
    extra_o = octets % _NS
    r0 = (sid * base_o + jnp.minimum(sid, extra_o)) * 8
    my_rows = (base_o + jnp.where(sid < extra_o, 1, 0)) * 8
    zrows = zbuf.shape[0]
    n_big = (base_o * 8) // zrows
    n_small = (my_rows - n_big * zrows) // 8

    iota = lax.iota(jnp.int32, _L)
    zvec = jnp.zeros((_L,), jnp.float32)
    onevec = jnp.ones((_L,), jnp.float32)

    def _zfill(i, _):
        zbuf[i] = zvec
        return 0
    lax.fori_loop(0, zrows, _zfill, 0)
    for j in range(n_big):
        pltpu.sync_copy(zbuf, acc_sh.at[pl.ds(r0 + j * zrows, zrows)])

    def _ztail(j, _):
        pltpu.sync_copy(
            zbuf.at[pl.ds(0, 8)],
            acc_sh.at[pl.ds(r0 + n_big * zrows + j * 8, 8)])
        return 0
    lax.fori_loop(0, n_small, _ztail, 0)

    def _oinit(g, _):
        r = g * _L + iota
        plsc.store_scatter(outbuf, [r, jnp.full((_L,), 9, jnp.int32)], onevec)
        for c in range(10, 16):
            plsc.store_scatter(outbuf, [r, jnp.full((_L,), c, jnp.int32)], zvec)
        return 0
    lax.fori_loop(0, 8, _oinit, 0)

    plsc.subcore_barrier()

    base = n_groups_total // _NW
    extra = n_groups_total % _NW
    my_g = base + jnp.where(wid < extra, 1, 0)
    g0 = wid * base + jnp.minimum(wid, extra)

    def _edge_group(t, _):
        e0 = (g0 + t) * 128
        pltpu.sync_copy(row_hbm.at[pl.ds(e0, 128)], rbuf)
        pltpu.sync_copy(col_hbm.at[pl.ds(e0, 128)], cbuf)
        pltpu.sync_copy(fr_hbm.at[pl.ds(e0 * 9, 1152)], fbuf)
        pltpu.async_copy(vdf_hbm.at[rbuf], vgbuf, sem).wait()

        def _grp(g, _):
            r = g * _L + iota
            fb = g * (_L * 9) + iota * 9
            f = [plsc.load_gather(fbuf, [fb + c]) for c in range(9)]
            v = [plsc.load_gather(vgbuf, [r, jnp.full((_L,), c, jnp.int32)])
                 for c in range(9)]
            for s in range(3):
                for i in range(3):
                    acc = (f[i * 3] * v[s * 3]
                           + f[i * 3 + 1] * v[s * 3 + 1]
                           + f[i * 3 + 2] * v[s * 3 + 2])
                    plsc.store_scatter(
                        outbuf, [r, jnp.full((_L,), s * 3 + i, jnp.int32)], acc)
            return 0
        lax.fori_loop(0, 8, _grp, 0)

        pltpu.sync_copy(outbuf, acc_sh.at[cbuf], add=True)
        return 0
    lax.fori_loop(0, my_g, _edge_group, 0)

    plsc.subcore_barrier()

    for j in range(n_big):
        pltpu.sync_copy(
            acc_sh.at[pl.ds(r0 + j * zrows, zrows)],
            out_hbm.at[cid, pl.ds(r0 + j * zrows, zrows)])

    def _dtail(j, _):
        off = r0 + n_big * zrows + j * 8
        pltpu.sync_copy(acc_sh.at[pl.ds(off, 8)],
                        out_hbm.at[cid, pl.ds(off, 8)])
        return 0
    lax.fori_loop(0, n_small, _dtail, 0)


def _sc_scalarize(row, col, frames_flat, vdf_pad):
    n_nodes = vdf_pad.shape[0]
    n_groups = row.shape[0] // 128
    mesh = plsc.VectorSubcoreMesh(
        core_axis_name="c", subcore_axis_name="s",
        num_cores=_NC, num_subcores=_NS)
    fn = pl.kernel(
        functools.partial(_sc_body, n_nodes, n_groups),
        out_type=jax.ShapeDtypeStruct((_NC, n_nodes, 16), jnp.float32),
        mesh=mesh,
        compiler_params=pltpu.CompilerParams(
            needs_layout_passes=False, use_tc_tiling_on_sc=False),
        scratch_types=[
            pltpu.VMEM((128,), jnp.int32),
            pltpu.VMEM((128,), jnp.int32),
            pltpu.VMEM((1152,), jnp.float32),
            pltpu.VMEM((128, 16), jnp.float32),
            pltpu.VMEM((128, 16), jnp.float32),
            pltpu.VMEM((1000, 16), jnp.float32),
            pltpu.VMEM_SHARED((n_nodes, 16), jnp.float32),
            pltpu.SemaphoreType.DMA,
        ],
    )
    return fn(row, col, frames_flat, vdf_pad)


def _dense_body(sr_ref, vr_ref, a0_ref, a1_ref, wvd_ref, msum_ref, wss_ref,
                wsn_ref, wsm_ref, bso_ref, wvos_ref, bvos_ref, wvu_ref,
                s_ref, v_ref):
    dot = functools.partial(
        lax.dot_general, precision=_PREC, preferred_element_type=jnp.float32)
    mm = lambda a, b: dot(a, b, (((1,), (0,)), ((), ())))

    vr2 = vr_ref[...]
    vh2 = mm(vr2, wvd_ref[...])
    n2 = mm(vh2 * vh2, msum_ref[...]) + _EPS
    vn = jnp.sqrt(n2)

    acc = a0_ref[...] + a1_ref[...]
    lane = lax.broadcasted_iota(jnp.int32, acc.shape, 1)
    cnt = jnp.sum(jnp.where(lane == 9, acc, 0.0), axis=1, keepdims=True)
    mean16 = acc * (1.0 / jnp.maximum(cnt, 1.0))

    s_out = (mm(sr_ref[...], wss_ref[...]) + mm(vn, wsn_ref[...])
             + mm(mean16, wsm_ref[...]) + bso_ref[...])
    sf = s_out * jax.nn.sigmoid(s_out)

    gate48 = mm(sf, wvos_ref[...]) + bvos_ref[...]
    vu2 = mm(vh2, wvu_ref[...])

    s_ref[...] = sf
    v_ref[...] = vu2 * jax.nn.sigmoid(gate48)


def _dense_stage(sr, vr2, a0, a1, wvd2, msum, wss, wsn, wsm, bso, wvos48,
                 bvos48, wvu2, blk):
    n = sr.shape[0]
    full = lambda r, c: pl.BlockSpec((r, c), lambda i: (0, 0))
    return pl.pallas_call(
        _dense_body,
        grid=(n // blk,),
        in_specs=[
            pl.BlockSpec((blk, 128), lambda i: (i, 0)),
            pl.BlockSpec((blk, 48), lambda i: (i, 0)),
            pl.BlockSpec((blk, 16), lambda i: (i, 0)),
            pl.BlockSpec((blk, 16), lambda i: (i, 0)),
            full(48, 48), full(48, 16), full(128, 128), full(16, 128),
            full(16, 128), full(1, 128), full(128, 48), full(1, 48),
            full(48, 48),
        ],
        out_specs=[
            pl.BlockSpec((blk, 128), lambda i: (i, 0)),
            pl.BlockSpec((blk, 48), lambda i: (i, 0)),
        ],
        out_shape=[
            jax.ShapeDtypeStruct((n, 128), jnp.float32),
            jax.ShapeDtypeStruct((n, 48), jnp.float32),
        ],
    )(sr, vr2, a0, a1, wvd2, msum, wss, wsn, wsm, bso, wvos48, bvos48, wvu2)


def kernel(scalar_rep, vector_rep, edge_index, frames, W_vd, W_vdf, W_so,
           b_so, W_vu, W_vos, b_vos):
    n, s_in = scalar_rep.shape
    v_in = vector_rep.shape[1]
    hid = W_vd.shape[1]
    svo = W_vdf.shape[1]
    v_out = W_vu.shape[1]
    s_out_dim = W_so.shape[1]

    eye3 = jnp.eye(3, dtype=jnp.float32)
    w1 = jnp.einsum("vs,ab->vasb", W_vdf, eye3).reshape(3 * v_in, 3 * svo)
    w1p = jnp.pad(w1, ((0, 0), (0, 16 - 3 * svo)))
    wvd2 = jnp.einsum("vh,xy->vxyh", W_vd, eye3).reshape(3 * v_in, 3 * hid)
    msum = jnp.tile(jnp.eye(hid, dtype=jnp.float32), (3, 1))
    wvu2 = jnp.einsum("ho,xy->xhoy", W_vu, eye3).reshape(3 * hid, 3 * v_out)

    wss = W_so[:s_in]
    wsn = W_so[s_in:s_in + hid]
    wsm = jnp.pad(W_so[s_in + hid:], ((0, 16 - 3 * svo), (0, 0)))
    bso = b_so.reshape(1, s_out_dim)
    wvos48 = jnp.repeat(W_vos, 3, axis=1)
    bvos48 = jnp.repeat(b_vos, 3).reshape(1, 3 * v_out)

    vr2 = vector_rep.reshape(n, 3 * v_in)
    row = edge_index[0].astype(jnp.int32)
    col = edge_index[1].astype(jnp.int32)
    frames_flat = frames.reshape(-1)

    vdf_pad = _vdf_stage(vr2, w1p, blk=2000)
    acc = _sc_scalarize(row, col, frames_flat, vdf_pad)
    sf, vout48 = _dense_stage(
        scalar_rep, vr2, acc[0], acc[1], wvd2, msum, wss, wsn, wsm, bso,
        wvos48, bvos48, wvu2, blk=2000)
    return (sf, vout48.reshape(n, v_out, 3))


# speedup vs baseline: 3.4465x; 2.5737x over previous
import functools

import jax
import jax.numpy as jnp
from jax import lax
from jax.experimental import pallas as pl
from jax.experimental.pallas import tpu as pltpu
from jax.experimental.pallas import tpu_sc as plsc

_NC = 2
_NS = 16
_NW = _NC * _NS
_L = 16

_EPS = 1e-8
_PREC = lax.Precision.HIGHEST


def _ftr_body(fr_ref, out_ref):
    fr = fr_ref[...]
    blk = fr.shape[0]
    fr2 = jnp.concatenate(
        [fr[:, i, :] for i in range(3)]
        + [jnp.zeros((blk, 7), jnp.float32)], axis=1)
    out_ref[...] = fr2.T


def _ftr_stage(frames, blk):
    e = frames.shape[0]
    return pl.pallas_call(
        _ftr_body,
        grid=(e // blk,),
        in_specs=[pl.BlockSpec((blk, 3, 3), lambda i: (i, 0, 0))],
        out_specs=pl.BlockSpec((16, blk), lambda i: (0, i)),
        out_shape=jax.ShapeDtypeStruct((16, e), jnp.float32),
    )(frames)


def _vdf_body(vr_ref, w_ref, out_ref):
    out_ref[...] = lax.dot_general(
        vr_ref[...], w_ref[...], (((1,), (0,)), ((), ())),
        preferred_element_type=jnp.float32, precision=_PREC)


def _vdf_stage(vr2, w1p, blk):
    n = vr2.shape[0]
    return pl.pallas_call(
        _vdf_body,
        grid=(n // blk,),
        in_specs=[
            pl.BlockSpec((blk, 48), lambda i: (i, 0)),
            pl.BlockSpec((48, 16), lambda i: (0, 0)),
        ],
        out_specs=pl.BlockSpec((blk, 16), lambda i: (i, 0)),
        out_shape=jax.ShapeDtypeStruct((n, 16), jnp.float32),
    )(vr2, w1p)


_GPC = 5


def _sc_body(n_nodes, n_groups_total, row2_hbm, col2_hbm, fr9_hbm, vdf_hbm,
             out_hbm, rbuf, cbuf, fbuf9, vgbuf, outbuf, zbuf, acc_sh,
             sem_in, sem_g, sem_s):
    cid = lax.axis_index("c")
    sid = lax.axis_index("s")
    wid = sid * _NC + cid

    octets = n_nodes // 8
    base_o = octets // _NS
    extra_o = octets % _NS
    r0 = (sid * base_o + jnp.minimum(sid, extra_o)) * 8
    my_rows = (base_o + jnp.where(sid < extra_o, 1, 0)) * 8
    zrows = zbuf.shape[0]
    n_big = (base_o * 8) // zrows
    n_small = (my_rows - n_big * zrows) // 8

    iota = lax.iota(jnp.int32, _L)
    zvec = jnp.zeros((_L,), jnp.float32)
    onevec = jnp.ones((_L,), jnp.float32)

    def _zfill(i, _):
        zbuf[i] = zvec
        return 0
    lax.fori_loop(0, zrows, _zfill, 0)
    for j in range(n_big):
        pltpu.sync_copy(zbuf, acc_sh.at[pl.ds(r0 + j * zrows, zrows)])

    def _ztail(j, _):
        pltpu.sync_copy(
            zbuf.at[pl.ds(0, 8)],
            acc_sh.at[pl.ds(r0 + n_big * zrows + j * 8, 8)])
        return 0
    lax.fori_loop(0, n_small, _ztail, 0)

    def _oinit(g, _):
        r = g * _L + iota
        plsc.store_scatter(outbuf, [r, jnp.full((_L,), 9, jnp.int32)], onevec)
        for c in range(10, 16):
            plsc.store_scatter(outbuf, [r, jnp.full((_L,), c, jnp.int32)], zvec)
        return 0
    lax.fori_loop(0, _GPC * 8, _oinit, 0)

    plsc.subcore_barrier()

    base = n_groups_total // _NW
    extra = n_groups_total % _NW
    my_g = base + jnp.where(wid < extra, 1, 0)
    g0 = wid * base + jnp.minimum(wid, extra)
    n_chunks = my_g // _GPC
    n_tail = my_g - n_chunks * _GPC

    def _compute_group(g):
        r = g * _L + iota
        f = [fbuf9[c, pl.ds(g * _L, _L)] for c in range(9)]
        v = [plsc.load_gather(vgbuf, [r, jnp.full((_L,), c, jnp.int32)])
             for c in range(9)]
        for s in range(3):
            for i in range(3):
                acc = (f[i * 3] * v[s * 3]
                       + f[i * 3 + 1] * v[s * 3 + 1]
                       + f[i * 3 + 2] * v[s * 3 + 2])
                plsc.store_scatter(
                    outbuf, [r, jnp.full((_L,), s * 3 + i, jnp.int32)], acc)

    def _chunk(t, _):
        gidx = g0 + t * _GPC
        e0 = gidx * 128
        din = [pltpu.async_copy(row2_hbm.at[pl.ds(gidx, _GPC)], rbuf, sem_in),
               pltpu.async_copy(col2_hbm.at[pl.ds(gidx, _GPC)], cbuf, sem_in)]
        din += [pltpu.async_copy(
                    fr9_hbm.at[p, pl.ds(e0, _GPC * 128)], fbuf9.at[p], sem_in)
                for p in range(9)]
        for d in din:
            d.wait()
        dg = [pltpu.async_copy(
                  vdf_hbm.at[rbuf.at[j]],
                  vgbuf.at[pl.ds(j * 128, 128)], sem_g)
              for j in range(_GPC)]
        for d in dg:
            d.wait()

        def _grp(g, _):
            _compute_group(g)
            return 0
        lax.fori_loop(0, _GPC * 8, _grp, 0)

        ds_ = [pltpu.async_copy(
                   outbuf.at[pl.ds(j * 128, 128)],
                   acc_sh.at[cbuf.at[j]], sem_s, add=True)
               for j in range(_GPC)]
        for d in ds_:
            d.wait()
        return 0
    lax.fori_loop(0, n_chunks, _chunk, 0)

    def _tail_group(j, _):
        gidx = g0 + n_chunks * _GPC + j
        e0 = gidx * 128
        pltpu.sync_copy(row2_hbm.at[pl.ds(gidx, 1)], rbuf.at[pl.ds(0, 1)])
        pltpu.sync_copy(col2_hbm.at[pl.ds(gidx, 1)], cbuf.at[pl.ds(0, 1)])
        for p in range(9):
            pltpu.sync_copy(fr9_hbm.at[p, pl.ds(e0, 128)],
                            fbuf9.at[p, pl.ds(0, 128)])
        pltpu.async_copy(vdf_hbm.at[rbuf.at[0]],
                         vgbuf.at[pl.ds(0, 128)], sem_g).wait()

        def _grp(g, _):
            _compute_group(g)
            return 0
        lax.fori_loop(0, 8, _grp, 0)
        pltpu.sync_copy(outbuf.at[pl.ds(0, 128)], acc_sh.at[cbuf.at[0]],
                        add=True)
        return 0
    lax.fori_loop(0, n_tail, _tail_group, 0)

    plsc.subcore_barrier()

    for j in range(n_big):
        pltpu.sync_copy(
            acc_sh.at[pl.ds(r0 + j * zrows, zrows)],
            out_hbm.at[cid, pl.ds(r0 + j * zrows, zrows)])

    def _dtail(j, _):
        off = r0 + n_big * zrows + j * 8
        pltpu.sync_copy(acc_sh.at[pl.ds(off, 8)],
                        out_hbm.at[cid, pl.ds(off, 8)])
        return 0
    lax.fori_loop(0, n_small, _dtail, 0)


def _sc_scalarize(row2, col2, fr9, vdf_pad):
    n_nodes = vdf_pad.shape[0]
    n_groups = row2.shape[0]
    mesh = plsc.VectorSubcoreMesh(
        core_axis_name="c", subcore_axis_name="s",
        num_cores=_NC, num_subcores=_NS)
    fn = pl.kernel(
        functools.partial(_sc_body, n_nodes, n_groups),
        out_type=jax.ShapeDtypeStruct((_NC, n_nodes, 16), jnp.float32),
        mesh=mesh,
        compiler_params=pltpu.CompilerParams(
            needs_layout_passes=False, use_tc_tiling_on_sc=False),
        scratch_types=[
            pltpu.VMEM((_GPC, 128), jnp.int32),
            pltpu.VMEM((_GPC, 128), jnp.int32),
            pltpu.VMEM((9, _GPC * 128), jnp.float32),
            pltpu.VMEM((_GPC * 128, 16), jnp.float32),
            pltpu.VMEM((_GPC * 128, 16), jnp.float32),
            pltpu.VMEM((104, 16), jnp.float32),
            pltpu.VMEM_SHARED((n_nodes, 16), jnp.float32),
            pltpu.SemaphoreType.DMA,
            pltpu.SemaphoreType.DMA,
            pltpu.SemaphoreType.DMA,
        ],
    )
    return fn(row2, col2, fr9, vdf_pad)


def _dense_body(sr_ref, vr_ref, a0_ref, a1_ref, wvd_ref, msum_ref, wss_ref,
                wsn_ref, wsm_ref, bso_ref, wvos_ref, bvos_ref, wvu_ref,
                s_ref, v_ref):
    dot = functools.partial(
        lax.dot_general, precision=_PREC, preferred_element_type=jnp.float32)
    mm = lambda a, b: dot(a, b, (((1,), (0,)), ((), ())))

    vr2 = vr_ref[...]
    vh2 = mm(vr2, wvd_ref[...])
    n2 = mm(vh2 * vh2, msum_ref[...]) + _EPS
    vn = jnp.sqrt(n2)

    acc = a0_ref[...] + a1_ref[...]
    lane = lax.broadcasted_iota(jnp.int32, acc.shape, 1)
    cnt = jnp.sum(jnp.where(lane == 9, acc, 0.0), axis=1, keepdims=True)
    mean16 = acc * (1.0 / jnp.maximum(cnt, 1.0))

    s_out = (mm(sr_ref[...], wss_ref[...]) + mm(vn, wsn_ref[...])
             + mm(mean16, wsm_ref[...]) + bso_ref[...])
    sf = s_out * jax.nn.sigmoid(s_out)

    gate48 = mm(sf, wvos_ref[...]) + bvos_ref[...]
    vu2 = mm(vh2, wvu_ref[...])

    s_ref[...] = sf
    v_ref[...] = vu2 * jax.nn.sigmoid(gate48)


def _dense_stage(sr, vr2, a0, a1, wvd2, msum, wss, wsn, wsm, bso, wvos48,
                 bvos48, wvu2, blk):
    n = sr.shape[0]
    full = lambda r, c: pl.BlockSpec((r, c), lambda i: (0, 0))
    return pl.pallas_call(
        _dense_body,
        grid=(n // blk,),
        in_specs=[
            pl.BlockSpec((blk, 128), lambda i: (i, 0)),
            pl.BlockSpec((blk, 48), lambda i: (i, 0)),
            pl.BlockSpec((blk, 16), lambda i: (i, 0)),
            pl.BlockSpec((blk, 16), lambda i: (i, 0)),
            full(48, 48), full(48, 16), full(128, 128), full(16, 128),
            full(16, 128), full(1, 128), full(128, 48), full(1, 48),
            full(48, 48),
        ],
        out_specs=[
            pl.BlockSpec((blk, 128), lambda i: (i, 0)),
            pl.BlockSpec((blk, 48), lambda i: (i, 0)),
        ],
        out_shape=[
            jax.ShapeDtypeStruct((n, 128), jnp.float32),
            jax.ShapeDtypeStruct((n, 48), jnp.float32),
        ],
    )(sr, vr2, a0, a1, wvd2, msum, wss, wsn, wsm, bso, wvos48, bvos48, wvu2)


def kernel(scalar_rep, vector_rep, edge_index, frames, W_vd, W_vdf, W_so,
           b_so, W_vu, W_vos, b_vos):
    n, s_in = scalar_rep.shape
    v_in = vector_rep.shape[1]
    hid = W_vd.shape[1]
    svo = W_vdf.shape[1]
    v_out = W_vu.shape[1]
    s_out_dim = W_so.shape[1]

    eye3 = jnp.eye(3, dtype=jnp.float32)
    w1 = jnp.einsum("vs,ab->vasb", W_vdf, eye3).reshape(3 * v_in, 3 * svo)
    w1p = jnp.pad(w1, ((0, 0), (0, 16 - 3 * svo)))
    wvd2 = jnp.einsum("vh,xy->vxyh", W_vd, eye3).reshape(3 * v_in, 3 * hid)
    msum = jnp.tile(jnp.eye(hid, dtype=jnp.float32), (3, 1))
    wvu2 = jnp.einsum("ho,xy->xhoy", W_vu, eye3).reshape(3 * hid, 3 * v_out)

    wss = W_so[:s_in]
    wsn = W_so[s_in:s_in + hid]
    wsm = jnp.pad(W_so[s_in + hid:], ((0, 16 - 3 * svo), (0, 0)))
    bso = b_so.reshape(1, s_out_dim)
    wvos48 = jnp.repeat(W_vos, 3, axis=1)
    bvos48 = jnp.repeat(b_vos, 3).reshape(1, 3 * v_out)

    vr2 = vector_rep.reshape(n, 3 * v_in)
    e = frames.shape[0]
    row2 = edge_index[0].astype(jnp.int32).reshape(e // 128, 128)
    col2 = edge_index[1].astype(jnp.int32).reshape(e // 128, 128)

    fr9 = _ftr_stage(frames, blk=3200)
    vdf_pad = _vdf_stage(vr2, w1p, blk=2000)
    acc = _sc_scalarize(row2, col2, fr9, vdf_pad)
    sf, vout48 = _dense_stage(
        scalar_rep, vr2, acc[0], acc[1], wvd2, msum, wss, wsn, wsm, bso,
        wvos48, bvos48, wvu2, blk=2000)
    return (sf, vout48.reshape(n, v_out, 3))


# speedup vs baseline: 9.8316x; 2.8526x over previous
import functools

import jax
import jax.numpy as jnp
from jax import lax
from jax.experimental import pallas as pl
from jax.experimental.pallas import tpu as pltpu
from jax.experimental.pallas import tpu_sc as plsc

_NC = 2
_NS = 16
_NW = _NC * _NS
_L = 16

_EPS = 1e-8
_PREC = lax.Precision.HIGHEST


def _ftr_body(fr_ref, out_ref):
    fr = fr_ref[...]
    blk = fr.shape[0]
    fr2 = jnp.concatenate(
        [fr[:, i, :] for i in range(3)]
        + [jnp.zeros((blk, 7), jnp.float32)], axis=1)
    out_ref[...] = fr2.T


def _ftr_stage(frames, blk):
    e = frames.shape[0]
    return pl.pallas_call(
        _ftr_body,
        grid=(e // blk,),
        in_specs=[pl.BlockSpec((blk, 3, 3), lambda i: (i, 0, 0))],
        out_specs=pl.BlockSpec((16, blk), lambda i: (0, i)),
        out_shape=jax.ShapeDtypeStruct((16, e), jnp.float32),
    )(frames)


def _vdf_body(vr_ref, w_ref, out_ref):
    out_ref[...] = lax.dot_general(
        vr_ref[...], w_ref[...], (((1,), (0,)), ((), ())),
        preferred_element_type=jnp.float32, precision=_PREC)


def _vdf_stage(vr2, w1p, blk):
    n = vr2.shape[0]
    return pl.pallas_call(
        _vdf_body,
        grid=(n // blk,),
        in_specs=[
            pl.BlockSpec((blk, 48), lambda i: (i, 0)),
            pl.BlockSpec((48, 16), lambda i: (0, 0)),
        ],
        out_specs=pl.BlockSpec((blk, 16), lambda i: (i, 0)),
        out_shape=jax.ShapeDtypeStruct((n, 16), jnp.float32),
    )(vr2, w1p)


_GPC = 5


def _sc_body(n_nodes, n_groups_total, row2_hbm, col2_hbm, fr9_hbm, vdf_hbm,
             out_hbm, rbuf, cbuf, fbuf9, vgbuf, outbuf, zbuf, acc_sh,
             sem_in, sem_g, sem_s):
    cid = lax.axis_index("c")
    sid = lax.axis_index("s")
    wid = sid * _NC + cid

    octets = n_nodes // 8
    base_o = octets // _NS
    extra_o = octets % _NS
    r0 = (sid * base_o + jnp.minimum(sid, extra_o)) * 8
    my_rows = (base_o + jnp.where(sid < extra_o, 1, 0)) * 8
    zrows = zbuf.shape[0]
    n_big = (base_o * 8) // zrows
    n_small = (my_rows - n_big * zrows) // 8

    iota = lax.iota(jnp.int32, _L)
    zvec = jnp.zeros((_L,), jnp.float32)
    onevec = jnp.ones((_L,), jnp.float32)

    def _zfill(i, _):
        zbuf[i] = zvec
        return 0
    lax.fori_loop(0, zrows, _zfill, 0)
    for j in range(n_big):
        pltpu.sync_copy(zbuf, acc_sh.at[pl.ds(r0 + j * zrows, zrows)])

    def _ztail(j, _):
        pltpu.sync_copy(
            zbuf.at[pl.ds(0, 8)],
            acc_sh.at[pl.ds(r0 + n_big * zrows + j * 8, 8)])
        return 0
    lax.fori_loop(0, n_small, _ztail, 0)

    def _oinit(g, _):
        r = g * _L + iota
        plsc.store_scatter(outbuf, [r, jnp.full((_L,), 9, jnp.int32)], onevec)
        for c in range(10, 16):
            plsc.store_scatter(outbuf, [r, jnp.full((_L,), c, jnp.int32)], zvec)
        return 0
    lax.fori_loop(0, _GPC * 8, _oinit, 0)

    plsc.subcore_barrier()

    base = n_groups_total // _NW
    extra = n_groups_total % _NW
    my_g = base + jnp.where(wid < extra, 1, 0)
    g0 = wid * base + jnp.minimum(wid, extra)
    n_chunks = my_g // _GPC
    n_tail = my_g - n_chunks * _GPC

    def _compute_group(g):
        r = g * _L + iota
        f = [fbuf9[c, pl.ds(g * _L, _L)] for c in range(9)]
        v = [plsc.load_gather(vgbuf, [r, jnp.full((_L,), c, jnp.int32)])
             for c in range(9)]
        for s in range(3):
            for i in range(3):
                acc = (f[i * 3] * v[s * 3]
                       + f[i * 3 + 1] * v[s * 3 + 1]
                       + f[i * 3 + 2] * v[s * 3 + 2])
                plsc.store_scatter(
                    outbuf, [r, jnp.full((_L,), s * 3 + i, jnp.int32)], acc)

    def _chunk(t, _):
        gidx = g0 + t * _GPC
        e0 = gidx * 128
        din = [pltpu.async_copy(row2_hbm.at[pl.ds(gidx, _GPC)], rbuf, sem_in),
               pltpu.async_copy(col2_hbm.at[pl.ds(gidx, _GPC)], cbuf, sem_in)]
        din += [pltpu.async_copy(
                    fr9_hbm.at[p, pl.ds(e0, _GPC * 128)], fbuf9.at[p], sem_in)
                for p in range(9)]
        for d in din:
            d.wait()
        dg = [pltpu.async_copy(
                  vdf_hbm.at[rbuf.at[j]],
                  vgbuf.at[pl.ds(j * 128, 128)], sem_g)
              for j in range(_GPC)]
        for d in dg:
            d.wait()

        def _grp(g, _):
            _compute_group(g)
            return 0
        lax.fori_loop(0, _GPC * 8, _grp, 0)

        ds_ = [pltpu.async_copy(
                   outbuf.at[pl.ds(j * 128, 128)],
                   acc_sh.at[cbuf.at[j]], sem_s, add=True)
               for j in range(_GPC)]
        for d in ds_:
            d.wait()
        return 0
    lax.fori_loop(0, n_chunks, _chunk, 0)

    def _tail_group(j, _):
        gidx = g0 + n_chunks * _GPC + j
        e0 = gidx * 128
        pltpu.sync_copy(row2_hbm.at[pl.ds(gidx, 1)], rbuf.at[pl.ds(0, 1)])
        pltpu.sync_copy(col2_hbm.at[pl.ds(gidx, 1)], cbuf.at[pl.ds(0, 1)])
        for p in range(9):
            pltpu.sync_copy(fr9_hbm.at[p, pl.ds(e0, 128)],
                            fbuf9.at[p, pl.ds(0, 128)])
        pltpu.async_copy(vdf_hbm.at[rbuf.at[0]],
                         vgbuf.at[pl.ds(0, 128)], sem_g).wait()

        def _grp(g, _):
            _compute_group(g)
            return 0
        lax.fori_loop(0, 8, _grp, 0)
        pltpu.sync_copy(outbuf.at[pl.ds(0, 128)], acc_sh.at[cbuf.at[0]],
                        add=True)
        return 0
    lax.fori_loop(0, n_tail, _tail_group, 0)

    plsc.subcore_barrier()

    for j in range(n_big):
        pltpu.sync_copy(
            acc_sh.at[pl.ds(r0 + j * zrows, zrows)],
            out_hbm.at[cid, pl.ds(r0 + j * zrows, zrows)])

    def _dtail(j, _):
        off = r0 + n_big * zrows + j * 8
        pltpu.sync_copy(acc_sh.at[pl.ds(off, 8)],
                        out_hbm.at[cid, pl.ds(off, 8)])
        return 0
    lax.fori_loop(0, n_small, _dtail, 0)


def _sc_scalarize(row2, col2, fr9, vdf_pad):
    n_nodes = vdf_pad.shape[0]
    n_groups = row2.shape[0]
    mesh = plsc.VectorSubcoreMesh(
        core_axis_name="c", subcore_axis_name="s",
        num_cores=_NC, num_subcores=_NS)
    fn = pl.kernel(
        functools.partial(_sc_body, n_nodes, n_groups),
        out_type=jax.ShapeDtypeStruct((_NC, n_nodes, 16), jnp.float32),
        mesh=mesh,
        compiler_params=pltpu.CompilerParams(
            needs_layout_passes=False, use_tc_tiling_on_sc=False),
        scratch_types=[
            pltpu.VMEM((_GPC, 128), jnp.int32),
            pltpu.VMEM((_GPC, 128), jnp.int32),
            pltpu.VMEM((9, _GPC * 128), jnp.float32),
            pltpu.VMEM((_GPC * 128, 16), jnp.float32),
            pltpu.VMEM((_GPC * 128, 16), jnp.float32),
            pltpu.VMEM((104, 16), jnp.float32),
            pltpu.VMEM_SHARED((n_nodes, 16), jnp.float32),
            pltpu.SemaphoreType.DMA,
            pltpu.SemaphoreType.DMA,
            pltpu.SemaphoreType.DMA,
        ],
    )
    return fn(row2, col2, fr9, vdf_pad)


def _dense_body(sr_ref, vr_ref, a0_ref, a1_ref, wvd_ref, msum_ref, wss_ref,
                wsn_ref, wsm_ref, bso_ref, wvos_ref, bvos_ref, wvu_ref,
                s_ref, v_ref):
    dot = functools.partial(
        lax.dot_general, precision=_PREC, preferred_element_type=jnp.float32)
    mm = lambda a, b: dot(a, b, (((1,), (0,)), ((), ())))

    vr2 = vr_ref[...]
    vh2 = mm(vr2, wvd_ref[...])
    n2 = mm(vh2 * vh2, msum_ref[...]) + _EPS
    vn = jnp.sqrt(n2)

    acc = a0_ref[...] + a1_ref[...]
    lane = lax.broadcasted_iota(jnp.int32, acc.shape, 1)
    cnt = jnp.sum(jnp.where(lane == 9, acc, 0.0), axis=1, keepdims=True)
    mean16 = acc * (1.0 / jnp.maximum(cnt, 1.0))

    s_out = (mm(sr_ref[...], wss_ref[...]) + mm(vn, wsn_ref[...])
             + mm(mean16, wsm_ref[...]) + bso_ref[...])
    sf = s_out * jax.nn.sigmoid(s_out)

    gate48 = mm(sf, wvos_ref[...]) + bvos_ref[...]
    vu2 = mm(vh2, wvu_ref[...])

    s_ref[...] = sf
    v_ref[...] = vu2 * jax.nn.sigmoid(gate48)


def _dense_stage(sr, vr2, a0, a1, wvd2, msum, wss, wsn, wsm, bso, wvos48,
                 bvos48, wvu2, blk):
    n = sr.shape[0]
    full = lambda r, c: pl.BlockSpec((r, c), lambda i: (0, 0))
    return pl.pallas_call(
        _dense_body,
        grid=(n // blk,),
        in_specs=[
            pl.BlockSpec((blk, 128), lambda i: (i, 0)),
            pl.BlockSpec((blk, 48), lambda i: (i, 0)),
            pl.BlockSpec((blk, 16), lambda i: (i, 0)),
            pl.BlockSpec((blk, 16), lambda i: (i, 0)),
            full(48, 48), full(48, 16), full(128, 128), full(16, 128),
            full(16, 128), full(1, 128), full(128, 48), full(1, 48),
            full(48, 48),
        ],
        out_specs=[
            pl.BlockSpec((blk, 128), lambda i: (i, 0)),
            pl.BlockSpec((blk, 48), lambda i: (i, 0)),
        ],
        out_shape=[
            jax.ShapeDtypeStruct((n, 128), jnp.float32),
            jax.ShapeDtypeStruct((n, 48), jnp.float32),
        ],
    )(sr, vr2, a0, a1, wvd2, msum, wss, wsn, wsm, bso, wvos48, bvos48, wvu2)


def kernel(scalar_rep, vector_rep, edge_index, frames, W_vd, W_vdf, W_so,
           b_so, W_vu, W_vos, b_vos):
    n, s_in = scalar_rep.shape
    v_in = vector_rep.shape[1]
    hid = W_vd.shape[1]
    svo = W_vdf.shape[1]
    v_out = W_vu.shape[1]
    s_out_dim = W_so.shape[1]

    eye3 = jnp.eye(3, dtype=jnp.float32)
    w1 = jnp.einsum("vs,ab->vasb", W_vdf, eye3).reshape(3 * v_in, 3 * svo)
    w1p = jnp.pad(w1, ((0, 0), (0, 16 - 3 * svo)))
    wvd2 = jnp.einsum("vh,xy->vxyh", W_vd, eye3).reshape(3 * v_in, 3 * hid)
    msum = jnp.tile(jnp.eye(hid, dtype=jnp.float32), (3, 1))
    wvu2 = jnp.einsum("ho,xy->xhoy", W_vu, eye3).reshape(3 * hid, 3 * v_out)

    wss = W_so[:s_in]
    wsn = W_so[s_in:s_in + hid]
    wsm = jnp.pad(W_so[s_in + hid:], ((0, 16 - 3 * svo), (0, 0)))
    bso = b_so.reshape(1, s_out_dim)
    wvos48 = jnp.repeat(W_vos, 3, axis=1)
    bvos48 = jnp.repeat(b_vos, 3).reshape(1, 3 * v_out)

    vr2 = vector_rep.reshape(n, 3 * v_in)
    e = frames.shape[0]
    row2 = edge_index[0].astype(jnp.int32).reshape(e // 128, 128)
    col2 = edge_index[1].astype(jnp.int32).reshape(e // 128, 128)

    fr9 = jnp.transpose(frames, (1, 2, 0)).reshape(9, e)
    vdf_pad = _vdf_stage(vr2, w1p, blk=2000)
    acc = _sc_scalarize(row2, col2, fr9, vdf_pad)
    sf, vout48 = _dense_stage(
        scalar_rep, vr2, acc[0], acc[1], wvd2, msum, wss, wsn, wsm, bso,
        wvos48, bvos48, wvu2, blk=2000)
    return (sf, vout48.reshape(n, v_out, 3))


# speedup vs baseline: 16.3045x; 1.6584x over previous
import functools

import jax
import jax.numpy as jnp
from jax import lax
from jax.experimental import pallas as pl
from jax.experimental.pallas import tpu as pltpu
from jax.experimental.pallas import tpu_sc as plsc

_NC = 2
_NS = 16
_NW = _NC * _NS
_L = 16

_EPS = 1e-8
_PREC = lax.Precision.HIGHEST


def _ftr_body(fr_ref, *out_refs):
    fr = fr_ref[...]
    blk = fr.shape[-1]
    for p, oref in enumerate(out_refs):
        i, k = divmod(p, 3)
        oref[...] = fr[i, k, :].reshape(blk // 128, 128)


def _ftr_stage(frames_t, blk):
    e = frames_t.shape[-1]
    br = blk // 128
    return pl.pallas_call(
        _ftr_body,
        grid=(e // blk,),
        in_specs=[pl.BlockSpec((3, 3, blk), lambda i: (0, 0, i))],
        out_specs=[pl.BlockSpec((br, 128), lambda i: (i, 0))
                   for _ in range(9)],
        out_shape=[jax.ShapeDtypeStruct((e // 128, 128), jnp.float32)
                   for _ in range(9)],
    )(frames_t)


def _vdf_body(vr_ref, w_ref, out_ref):
    out_ref[...] = lax.dot_general(
        vr_ref[...], w_ref[...], (((1,), (0,)), ((), ())),
        preferred_element_type=jnp.float32, precision=_PREC)


def _vdf_stage(vr2, w1p, blk):
    n = vr2.shape[0]
    return pl.pallas_call(
        _vdf_body,
        grid=(n // blk,),
        in_specs=[
            pl.BlockSpec((blk, 48), lambda i: (i, 0)),
            pl.BlockSpec((48, 16), lambda i: (0, 0)),
        ],
        out_specs=pl.BlockSpec((blk, 16), lambda i: (i, 0)),
        out_shape=jax.ShapeDtypeStruct((n, 16), jnp.float32),
    )(vr2, w1p)


_GPC = 5


def _sc_body(n_nodes, n_groups_total, row2_hbm, col2_hbm,
             f0, f1, f2, f3, f4, f5, f6, f7, f8, vdf_hbm,
             out_hbm, rbuf, cbuf, fbuf9, vgbuf, outbuf, zbuf, acc_sh,
             sem_in, sem_g, sem_s):
    frs = (f0, f1, f2, f3, f4, f5, f6, f7, f8)
    cid = lax.axis_index("c")
    sid = lax.axis_index("s")
    wid = sid * _NC + cid

    octets = n_nodes // 8
    base_o = octets // _NS
    extra_o = octets % _NS
    r0 = (sid * base_o + jnp.minimum(sid, extra_o)) * 8
    my_rows = (base_o + jnp.where(sid < extra_o, 1, 0)) * 8
    zrows = zbuf.shape[0]
    n_big = (base_o * 8) // zrows
    n_small = (my_rows - n_big * zrows) // 8

    iota = lax.iota(jnp.int32, _L)
    zvec = jnp.zeros((_L,), jnp.float32)
    onevec = jnp.ones((_L,), jnp.float32)

    def _zfill(i, _):
        zbuf[i] = zvec
        return 0
    lax.fori_loop(0, zrows, _zfill, 0)
    for j in range(n_big):
        pltpu.sync_copy(zbuf, acc_sh.at[pl.ds(r0 + j * zrows, zrows)])

    def _ztail(j, _):
        pltpu.sync_copy(
            zbuf.at[pl.ds(0, 8)],
            acc_sh.at[pl.ds(r0 + n_big * zrows + j * 8, 8)])
        return 0
    lax.fori_loop(0, n_small, _ztail, 0)

    def _oinit(g, _):
        r = g * _L + iota
        plsc.store_scatter(outbuf, [r, jnp.full((_L,), 9, jnp.int32)], onevec)
        for c in range(10, 16):
            plsc.store_scatter(outbuf, [r, jnp.full((_L,), c, jnp.int32)], zvec)
        return 0
    lax.fori_loop(0, _GPC * 8, _oinit, 0)

    plsc.subcore_barrier()

    base = n_groups_total // _NW
    extra = n_groups_total % _NW
    my_g = base + jnp.where(wid < extra, 1, 0)
    g0 = wid * base + jnp.minimum(wid, extra)
    n_chunks = my_g // _GPC
    n_tail = my_g - n_chunks * _GPC

    def _compute_group(g):
        r = g * _L + iota
        f = [fbuf9[c, g // 8, pl.ds((g % 8) * _L, _L)] for c in range(9)]
        v = [plsc.load_gather(vgbuf, [r, jnp.full((_L,), c, jnp.int32)])
             for c in range(9)]
        for s in range(3):
            for i in range(3):
                acc = (f[i * 3] * v[s * 3]
                       + f[i * 3 + 1] * v[s * 3 + 1]
                       + f[i * 3 + 2] * v[s * 3 + 2])
                plsc.store_scatter(
                    outbuf, [r, jnp.full((_L,), s * 3 + i, jnp.int32)], acc)

    def _chunk(t, _):
        gidx = g0 + t * _GPC
        e0 = gidx * 128
        din = [pltpu.async_copy(row2_hbm.at[pl.ds(gidx, _GPC)], rbuf, sem_in),
               pltpu.async_copy(col2_hbm.at[pl.ds(gidx, _GPC)], cbuf, sem_in)]
        din += [pltpu.async_copy(
                    frs[p].at[pl.ds(gidx, _GPC)], fbuf9.at[p], sem_in)
                for p in range(9)]
        for d in din:
            d.wait()
        dg = [pltpu.async_copy(
                  vdf_hbm.at[rbuf.at[j]],
                  vgbuf.at[pl.ds(j * 128, 128)], sem_g)
              for j in range(_GPC)]
        for d in dg:
            d.wait()

        def _grp(g, _):
            _compute_group(g)
            return 0
        lax.fori_loop(0, _GPC * 8, _grp, 0)

        ds_ = [pltpu.async_copy(
                   outbuf.at[pl.ds(j * 128, 128)],
                   acc_sh.at[cbuf.at[j]], sem_s, add=True)
               for j in range(_GPC)]
        for d in ds_:
            d.wait()
        return 0
    lax.fori_loop(0, n_chunks, _chunk, 0)

    def _tail_group(j, _):
        gidx = g0 + n_chunks * _GPC + j
        e0 = gidx * 128
        pltpu.sync_copy(row2_hbm.at[pl.ds(gidx, 1)], rbuf.at[pl.ds(0, 1)])
        pltpu.sync_copy(col2_hbm.at[pl.ds(gidx, 1)], cbuf.at[pl.ds(0, 1)])
        for p in range(9):
            pltpu.sync_copy(frs[p].at[pl.ds(gidx, 1)],
                            fbuf9.at[p, pl.ds(0, 1)])
        pltpu.async_copy(vdf_hbm.at[rbuf.at[0]],
                         vgbuf.at[pl.ds(0, 128)], sem_g).wait()

        def _grp(g, _):
            _compute_group(g)
            return 0
        lax.fori_loop(0, 8, _grp, 0)
        pltpu.sync_copy(outbuf.at[pl.ds(0, 128)], acc_sh.at[cbuf.at[0]],
                        add=True)
        return 0
    lax.fori_loop(0, n_tail, _tail_group, 0)

    plsc.subcore_barrier()

    for j in range(n_big):
        pltpu.sync_copy(
            acc_sh.at[pl.ds(r0 + j * zrows, zrows)],
            out_hbm.at[cid, pl.ds(r0 + j * zrows, zrows)])

    def _dtail(j, _):
        off = r0 + n_big * zrows + j * 8
        pltpu.sync_copy(acc_sh.at[pl.ds(off, 8)],
                        out_hbm.at[cid, pl.ds(off, 8)])
        return 0
    lax.fori_loop(0, n_small, _dtail, 0)


def _sc_scalarize(row2, col2, fr_planes, vdf_pad):
    n_nodes = vdf_pad.shape[0]
    n_groups = row2.shape[0]
    mesh = plsc.VectorSubcoreMesh(
        core_axis_name="c", subcore_axis_name="s",
        num_cores=_NC, num_subcores=_NS)
    fn = pl.kernel(
        functools.partial(_sc_body, n_nodes, n_groups),
        out_type=jax.ShapeDtypeStruct((_NC, n_nodes, 16), jnp.float32),
        mesh=mesh,
        compiler_params=pltpu.CompilerParams(
            needs_layout_passes=False, use_tc_tiling_on_sc=False),
        scratch_types=[
            pltpu.VMEM((_GPC, 128), jnp.int32),
            pltpu.VMEM((_GPC, 128), jnp.int32),
            pltpu.VMEM((9, _GPC, 128), jnp.float32),
            pltpu.VMEM((_GPC * 128, 16), jnp.float32),
            pltpu.VMEM((_GPC * 128, 16), jnp.float32),
            pltpu.VMEM((104, 16), jnp.float32),
            pltpu.VMEM_SHARED((n_nodes, 16), jnp.float32),
            pltpu.SemaphoreType.DMA,
            pltpu.SemaphoreType.DMA,
            pltpu.SemaphoreType.DMA,
        ],
    )
    return fn(row2, col2, *fr_planes, vdf_pad)


def _dense_body(sr_ref, vr_ref, a0_ref, a1_ref, wvd_ref, msum_ref, wss_ref,
                wsn_ref, wsm_ref, bso_ref, wvos_ref, bvos_ref, wvu_ref,
                s_ref, v_ref):
    dot = functools.partial(
        lax.dot_general, precision=_PREC, preferred_element_type=jnp.float32)
    mm = lambda a, b: dot(a, b, (((1,), (0,)), ((), ())))

    vr2 = vr_ref[...]
    vh2 = mm(vr2, wvd_ref[...])
    n2 = mm(vh2 * vh2, msum_ref[...]) + _EPS
    vn = jnp.sqrt(n2)

    acc = a0_ref[...] + a1_ref[...]
    lane = lax.broadcasted_iota(jnp.int32, acc.shape, 1)
    cnt = jnp.sum(jnp.where(lane == 9, acc, 0.0), axis=1, keepdims=True)
    mean16 = acc * (1.0 / jnp.maximum(cnt, 1.0))

    s_out = (mm(sr_ref[...], wss_ref[...]) + mm(vn, wsn_ref[...])
             + mm(mean16, wsm_ref[...]) + bso_ref[...])
    sf = s_out * jax.nn.sigmoid(s_out)

    gate48 = mm(sf, wvos_ref[...]) + bvos_ref[...]
    vu2 = mm(vh2, wvu_ref[...])

    s_ref[...] = sf
    v_ref[...] = vu2 * jax.nn.sigmoid(gate48)


def _dense_stage(sr, vr2, a0, a1, wvd2, msum, wss, wsn, wsm, bso, wvos48,
                 bvos48, wvu2, blk):
    n = sr.shape[0]
    full = lambda r, c: pl.BlockSpec((r, c), lambda i: (0, 0))
    return pl.pallas_call(
        _dense_body,
        grid=(n // blk,),
        in_specs=[
            pl.BlockSpec((blk, 128), lambda i: (i, 0)),
            pl.BlockSpec((blk, 48), lambda i: (i, 0)),
            pl.BlockSpec((blk, 16), lambda i: (i, 0)),
            pl.BlockSpec((blk, 16), lambda i: (i, 0)),
            full(48, 48), full(48, 16), full(128, 128), full(16, 128),
            full(16, 128), full(1, 128), full(128, 48), full(1, 48),
            full(48, 48),
        ],
        out_specs=[
            pl.BlockSpec((blk, 128), lambda i: (i, 0)),
            pl.BlockSpec((blk, 48), lambda i: (i, 0)),
        ],
        out_shape=[
            jax.ShapeDtypeStruct((n, 128), jnp.float32),
            jax.ShapeDtypeStruct((n, 48), jnp.float32),
        ],
    )(sr, vr2, a0, a1, wvd2, msum, wss, wsn, wsm, bso, wvos48, bvos48, wvu2)


def kernel(scalar_rep, vector_rep, edge_index, frames, W_vd, W_vdf, W_so,
           b_so, W_vu, W_vos, b_vos):
    n, s_in = scalar_rep.shape
    v_in = vector_rep.shape[1]
    hid = W_vd.shape[1]
    svo = W_vdf.shape[1]
    v_out = W_vu.shape[1]
    s_out_dim = W_so.shape[1]

    eye3 = jnp.eye(3, dtype=jnp.float32)
    w1 = jnp.einsum("vs,ab->vasb", W_vdf, eye3).reshape(3 * v_in, 3 * svo)
    w1p = jnp.pad(w1, ((0, 0), (0, 16 - 3 * svo)))
    wvd2 = jnp.einsum("vh,xy->vxyh", W_vd, eye3).reshape(3 * v_in, 3 * hid)
    msum = jnp.tile(jnp.eye(hid, dtype=jnp.float32), (3, 1))
    wvu2 = jnp.einsum("ho,xy->xhoy", W_vu, eye3).reshape(3 * hid, 3 * v_out)

    wss = W_so[:s_in]
    wsn = W_so[s_in:s_in + hid]
    wsm = jnp.pad(W_so[s_in + hid:], ((0, 16 - 3 * svo), (0, 0)))
    bso = b_so.reshape(1, s_out_dim)
    wvos48 = jnp.repeat(W_vos, 3, axis=1)
    bvos48 = jnp.repeat(b_vos, 3).reshape(1, 3 * v_out)

    vr2 = vector_rep.reshape(n, 3 * v_in)
    e = frames.shape[0]
    row2 = edge_index[0].astype(jnp.int32).reshape(e // 128, 128)
    col2 = edge_index[1].astype(jnp.int32).reshape(e // 128, 128)

    fr_t = jnp.pad(jnp.transpose(frames, (1, 2, 0)),
                   ((0, 0), (0, 0), (0, 5632)))
    fr9 = _ftr_stage(fr_t, blk=448 * 128)
    vdf_pad = _vdf_stage(vr2, w1p, blk=2000)
    acc = _sc_scalarize(row2, col2, fr9, vdf_pad)
    sf, vout48 = _dense_stage(
        scalar_rep, vr2, acc[0], acc[1], wvd2, msum, wss, wsn, wsm, bso,
        wvos48, bvos48, wvu2, blk=2000)
    return (sf, vout48.reshape(n, v_out, 3))


# speedup vs baseline: 22.8650x; 1.4024x over previous
import functools

import jax
import jax.numpy as jnp
from jax import lax
from jax.experimental import pallas as pl
from jax.experimental.pallas import tpu as pltpu
from jax.experimental.pallas import tpu_sc as plsc

_NC = 2
_NS = 16
_NW = _NC * _NS
_L = 16

_EPS = 1e-8
_PREC = lax.Precision.DEFAULT


def _ftr_body(fr_ref, *out_refs):
    fr = fr_ref[...]
    blk = fr.shape[-1]
    for p, oref in enumerate(out_refs):
        i, k = divmod(p, 3)
        oref[...] = fr[i, k, :].reshape(blk // 128, 128)


def _ftr_stage(frames_t, blk):
    e = frames_t.shape[-1]
    br = blk // 128
    return pl.pallas_call(
        _ftr_body,
        grid=(e // blk,),
        in_specs=[pl.BlockSpec((3, 3, blk), lambda i: (0, 0, i))],
        out_specs=[pl.BlockSpec((br, 128), lambda i: (i, 0))
                   for _ in range(9)],
        out_shape=[jax.ShapeDtypeStruct((e // 128, 128), jnp.float32)
                   for _ in range(9)],
    )(frames_t)


def _vdf_body(vr_ref, w_ref, out_ref):
    out_ref[...] = lax.dot_general(
        vr_ref[...], w_ref[...], (((1,), (0,)), ((), ())),
        preferred_element_type=jnp.float32, precision=_PREC)


def _vdf_stage(vr2, w1p, blk):
    n = vr2.shape[0]
    return pl.pallas_call(
        _vdf_body,
        grid=(n // blk,),
        in_specs=[
            pl.BlockSpec((blk, 48), lambda i: (i, 0)),
            pl.BlockSpec((48, 16), lambda i: (0, 0)),
        ],
        out_specs=pl.BlockSpec((blk, 16), lambda i: (i, 0)),
        out_shape=jax.ShapeDtypeStruct((n, 16), jnp.float32),
    )(vr2, w1p)


_GPC = 5


def _sc_body(n_nodes, n_groups_total, row2_hbm, col2_hbm,
             f0, f1, f2, f3, f4, f5, f6, f7, f8, vdf_hbm,
             out_hbm, rbuf, cbuf, fbuf9, vgbuf, outbuf, zbuf, acc_sh,
             sem_in, sem_g, sem_s):
    frs = (f0, f1, f2, f3, f4, f5, f6, f7, f8)
    cid = lax.axis_index("c")
    sid = lax.axis_index("s")
    wid = sid * _NC + cid

    octets = n_nodes // 8
    base_o = octets // _NS
    extra_o = octets % _NS
    r0 = (sid * base_o + jnp.minimum(sid, extra_o)) * 8
    my_rows = (base_o + jnp.where(sid < extra_o, 1, 0)) * 8
    zrows = zbuf.shape[0]
    n_big = (base_o * 8) // zrows
    n_small = (my_rows - n_big * zrows) // 8

    iota = lax.iota(jnp.int32, _L)
    zvec = jnp.zeros((_L,), jnp.float32)
    onevec = jnp.ones((_L,), jnp.float32)

    def _zfill(i, _):
        zbuf[i] = zvec
        return 0
    lax.fori_loop(0, zrows, _zfill, 0)
    for j in range(n_big):
        pltpu.sync_copy(zbuf, acc_sh.at[pl.ds(r0 + j * zrows, zrows)])

    def _ztail(j, _):
        pltpu.sync_copy(
            zbuf.at[pl.ds(0, 8)],
            acc_sh.at[pl.ds(r0 + n_big * zrows + j * 8, 8)])
        return 0
    lax.fori_loop(0, n_small, _ztail, 0)

    def _oinit(g, _):
        r = g * _L + iota
        plsc.store_scatter(outbuf, [r, jnp.full((_L,), 9, jnp.int32)], onevec)
        for c in range(10, 16):
            plsc.store_scatter(outbuf, [r, jnp.full((_L,), c, jnp.int32)], zvec)
        return 0
    lax.fori_loop(0, _GPC * 8, _oinit, 0)

    plsc.subcore_barrier()

    base = n_groups_total // _NW
    extra = n_groups_total % _NW
    my_g = base + jnp.where(wid < extra, 1, 0)
    g0 = wid * base + jnp.minimum(wid, extra)
    n_chunks = my_g // _GPC
    n_tail = my_g - n_chunks * _GPC

    def _compute_group(g):
        r = g * _L + iota
        f = [fbuf9[c, g // 8, pl.ds((g % 8) * _L, _L)] for c in range(9)]
        v = [plsc.load_gather(vgbuf, [r, jnp.full((_L,), c, jnp.int32)])
             for c in range(9)]
        for s in range(3):
            for i in range(3):
                acc = (f[i * 3] * v[s * 3]
                       + f[i * 3 + 1] * v[s * 3 + 1]
                       + f[i * 3 + 2] * v[s * 3 + 2])
                plsc.store_scatter(
                    outbuf, [r, jnp.full((_L,), s * 3 + i, jnp.int32)], acc)

    def _chunk(t, _):
        gidx = g0 + t * _GPC
        e0 = gidx * 128
        din = [pltpu.async_copy(row2_hbm.at[pl.ds(gidx, _GPC)], rbuf, sem_in),
               pltpu.async_copy(col2_hbm.at[pl.ds(gidx, _GPC)], cbuf, sem_in)]
        din += [pltpu.async_copy(
                    frs[p].at[pl.ds(gidx, _GPC)], fbuf9.at[p], sem_in)
                for p in range(9)]
        for d in din:
            d.wait()
        dg = [pltpu.async_copy(
                  vdf_hbm.at[rbuf.at[j]],
                  vgbuf.at[pl.ds(j * 128, 128)], sem_g)
              for j in range(_GPC)]
        for d in dg:
            d.wait()

        def _grp(g, _):
            _compute_group(g)
            return 0
        lax.fori_loop(0, _GPC * 8, _grp, 0)

        ds_ = [pltpu.async_copy(
                   outbuf.at[pl.ds(j * 128, 128)],
                   acc_sh.at[cbuf.at[j]], sem_s, add=True)
               for j in range(_GPC)]
        for d in ds_:
            d.wait()
        return 0
    lax.fori_loop(0, n_chunks, _chunk, 0)

    def _tail_group(j, _):
        gidx = g0 + n_chunks * _GPC + j
        e0 = gidx * 128
        pltpu.sync_copy(row2_hbm.at[pl.ds(gidx, 1)], rbuf.at[pl.ds(0, 1)])
        pltpu.sync_copy(col2_hbm.at[pl.ds(gidx, 1)], cbuf.at[pl.ds(0, 1)])
        for p in range(9):
            pltpu.sync_copy(frs[p].at[pl.ds(gidx, 1)],
                            fbuf9.at[p, pl.ds(0, 1)])
        pltpu.async_copy(vdf_hbm.at[rbuf.at[0]],
                         vgbuf.at[pl.ds(0, 128)], sem_g).wait()

        def _grp(g, _):
            _compute_group(g)
            return 0
        lax.fori_loop(0, 8, _grp, 0)
        pltpu.sync_copy(outbuf.at[pl.ds(0, 128)], acc_sh.at[cbuf.at[0]],
                        add=True)
        return 0
    lax.fori_loop(0, n_tail, _tail_group, 0)

    plsc.subcore_barrier()

    for j in range(n_big):
        pltpu.sync_copy(
            acc_sh.at[pl.ds(r0 + j * zrows, zrows)],
            out_hbm.at[cid, pl.ds(r0 + j * zrows, zrows)])

    def _dtail(j, _):
        off = r0 + n_big * zrows + j * 8
        pltpu.sync_copy(acc_sh.at[pl.ds(off, 8)],
                        out_hbm.at[cid, pl.ds(off, 8)])
        return 0
    lax.fori_loop(0, n_small, _dtail, 0)


def _sc_scalarize(row2, col2, fr_planes, vdf_pad):
    n_nodes = vdf_pad.shape[0]
    n_groups = row2.shape[0]
    mesh = plsc.VectorSubcoreMesh(
        core_axis_name="c", subcore_axis_name="s",
        num_cores=_NC, num_subcores=_NS)
    fn = pl.kernel(
        functools.partial(_sc_body, n_nodes, n_groups),
        out_type=jax.ShapeDtypeStruct((_NC, n_nodes, 16), jnp.float32),
        mesh=mesh,
        compiler_params=pltpu.CompilerParams(
            needs_layout_passes=False, use_tc_tiling_on_sc=False),
        scratch_types=[
            pltpu.VMEM((_GPC, 128), jnp.int32),
            pltpu.VMEM((_GPC, 128), jnp.int32),
            pltpu.VMEM((9, _GPC, 128), jnp.float32),
            pltpu.VMEM((_GPC * 128, 16), jnp.float32),
            pltpu.VMEM((_GPC * 128, 16), jnp.float32),
            pltpu.VMEM((104, 16), jnp.float32),
            pltpu.VMEM_SHARED((n_nodes, 16), jnp.float32),
            pltpu.SemaphoreType.DMA,
            pltpu.SemaphoreType.DMA,
            pltpu.SemaphoreType.DMA,
        ],
    )
    return fn(row2, col2, *fr_planes, vdf_pad)


def _dense_body(sr_ref, vr_ref, a0_ref, a1_ref, wvd_ref, msum_ref, wss_ref,
                wsn_ref, wsm_ref, bso_ref, wvos_ref, bvos_ref, wvu_ref,
                s_ref, v_ref):
    dot = functools.partial(
        lax.dot_general, precision=_PREC, preferred_element_type=jnp.float32)
    mm = lambda a, b: dot(a, b, (((1,), (0,)), ((), ())))

    vr2 = vr_ref[...]
    vh2 = mm(vr2, wvd_ref[...])
    n2 = mm(vh2 * vh2, msum_ref[...]) + _EPS
    vn = jnp.sqrt(n2)

    acc = a0_ref[...] + a1_ref[...]
    lane = lax.broadcasted_iota(jnp.int32, acc.shape, 1)
    cnt = jnp.sum(jnp.where(lane == 9, acc, 0.0), axis=1, keepdims=True)
    mean16 = acc * (1.0 / jnp.maximum(cnt, 1.0))

    s_out = (mm(sr_ref[...], wss_ref[...]) + mm(vn, wsn_ref[...])
             + mm(mean16, wsm_ref[...]) + bso_ref[...])
    sf = s_out * jax.nn.sigmoid(s_out)

    gate48 = mm(sf, wvos_ref[...]) + bvos_ref[...]
    vu2 = mm(vh2, wvu_ref[...])

    s_ref[...] = sf
    v_ref[...] = vu2 * jax.nn.sigmoid(gate48)


def _dense_stage(sr, vr2, a0, a1, wvd2, msum, wss, wsn, wsm, bso, wvos48,
                 bvos48, wvu2, blk):
    n = sr.shape[0]
    full = lambda r, c: pl.BlockSpec((r, c), lambda i: (0, 0))
    return pl.pallas_call(
        _dense_body,
        grid=(n // blk,),
        in_specs=[
            pl.BlockSpec((blk, 128), lambda i: (i, 0)),
            pl.BlockSpec((blk, 48), lambda i: (i, 0)),
            pl.BlockSpec((blk, 16), lambda i: (i, 0)),
            pl.BlockSpec((blk, 16), lambda i: (i, 0)),
            full(48, 48), full(48, 16), full(128, 128), full(16, 128),
            full(16, 128), full(1, 128), full(128, 48), full(1, 48),
            full(48, 48),
        ],
        out_specs=[
            pl.BlockSpec((blk, 128), lambda i: (i, 0)),
            pl.BlockSpec((blk, 48), lambda i: (i, 0)),
        ],
        out_shape=[
            jax.ShapeDtypeStruct((n, 128), jnp.float32),
            jax.ShapeDtypeStruct((n, 48), jnp.float32),
        ],
    )(sr, vr2, a0, a1, wvd2, msum, wss, wsn, wsm, bso, wvos48, bvos48, wvu2)


def kernel(scalar_rep, vector_rep, edge_index, frames, W_vd, W_vdf, W_so,
           b_so, W_vu, W_vos, b_vos):
    n, s_in = scalar_rep.shape
    v_in = vector_rep.shape[1]
    hid = W_vd.shape[1]
    svo = W_vdf.shape[1]
    v_out = W_vu.shape[1]
    s_out_dim = W_so.shape[1]

    eye3 = jnp.eye(3, dtype=jnp.float32)
    w1 = jnp.einsum("vs,ab->vasb", W_vdf, eye3).reshape(3 * v_in, 3 * svo)
    w1p = jnp.pad(w1, ((0, 0), (0, 16 - 3 * svo)))
    wvd2 = jnp.einsum("vh,xy->vxyh", W_vd, eye3).reshape(3 * v_in, 3 * hid)
    msum = jnp.tile(jnp.eye(hid, dtype=jnp.float32), (3, 1))
    wvu2 = jnp.einsum("ho,xy->xhoy", W_vu, eye3).reshape(3 * hid, 3 * v_out)

    wss = W_so[:s_in]
    wsn = W_so[s_in:s_in + hid]
    wsm = jnp.pad(W_so[s_in + hid:], ((0, 16 - 3 * svo), (0, 0)))
    bso = b_so.reshape(1, s_out_dim)
    wvos48 = jnp.repeat(W_vos, 3, axis=1)
    bvos48 = jnp.repeat(b_vos, 3).reshape(1, 3 * v_out)

    vr2 = vector_rep.reshape(n, 3 * v_in)
    e = frames.shape[0]
    row2 = edge_index[0].astype(jnp.int32).reshape(e // 128, 128)
    col2 = edge_index[1].astype(jnp.int32).reshape(e // 128, 128)

    fr_t = jnp.pad(jnp.transpose(frames, (1, 2, 0)),
                   ((0, 0), (0, 0), (0, 5632)))
    fr9 = _ftr_stage(fr_t, blk=448 * 128)
    vdf_pad = _vdf_stage(vr2, w1p, blk=2000)
    acc = _sc_scalarize(row2, col2, fr9, vdf_pad)
    sf, vout48 = _dense_stage(
        scalar_rep, vr2, acc[0], acc[1], wvd2, msum, wss, wsn, wsm, bso,
        wvos48, bvos48, wvu2, blk=2000)
    return (sf, vout48.reshape(n, v_out, 3))
